# baseline (device time: 365776 ns/iter reference)
import jax
import jax.numpy as jnp
from jax import lax
from jax.experimental import pallas as pl
from jax.experimental.pallas import tpu as pltpu

BN = 256
G = 4
N_SEND_SLOTS = 2


def _fused(A, Wo, S):
    S2, K = A.shape
    N = Wo.shape[1]
    NB = N // BN
    NG = NB // G
    GW = G * BN

    def body(a_ref, w_hbm, out_ref, w_buf, send_buf, recv_buf,
             w_sems, send_sems, recv_sems):
        my_x = lax.axis_index("x")
        my_y = lax.axis_index("y")
        nbr = (my_x, 1 - my_y)
        keep_off = my_y * S
        send_off = (1 - my_y) * S

        def w_copy(j, slot):
            return pltpu.make_async_copy(
                w_hbm.at[:, pl.ds(j * BN, BN)], w_buf.at[slot], w_sems.at[slot]
            )

        def exchange(ssl, g):
            return pltpu.make_async_remote_copy(
                src_ref=send_buf.at[ssl],
                dst_ref=recv_buf.at[g],
                send_sem=send_sems.at[ssl],
                recv_sem=recv_sems.at[g],
                device_id=nbr,
                device_id_type=pl.DeviceIdType.MESH,
            )

        w_copy(0, 0).start()

        barrier = pltpu.get_barrier_semaphore()
        pl.semaphore_signal(
            barrier, inc=1, device_id=nbr, device_id_type=pl.DeviceIdType.MESH
        )
        pl.semaphore_wait(barrier, 1)

        def step(j, carry):
            wslot = lax.rem(j, 2)
            g = lax.div(j, G)
            u = lax.rem(j, G)
            ssl = lax.rem(g, N_SEND_SLOTS)

            @pl.when(j + 1 < NB)
            def _():
                w_copy(j + 1, lax.rem(j + 1, 2)).start()

            w_copy(j, wslot).wait()
            w = w_buf[wslot].astype(jnp.bfloat16)

            p_s = jax.lax.dot(
                a_ref[pl.ds(send_off, S), :], w,
                preferred_element_type=jnp.float32,
            )

            @pl.when((u == 0) & (g >= N_SEND_SLOTS))
            def _():
                exchange(ssl, 0).wait_send()

            send_buf[ssl, :, pl.ds(u * BN, BN)] = p_s.astype(jnp.bfloat16)

            @pl.when(u == G - 1)
            def _():
                exchange(ssl, g).start()

            p_k = jax.lax.dot(
                a_ref[pl.ds(keep_off, S), :], w,
                preferred_element_type=jnp.float32,
            )
            out_ref[0, :, pl.ds(j * BN, BN)] = p_k.astype(jnp.bfloat16)

            @pl.when((u == G - 1) & (g >= 1))
            def _():
                gm = g - 1
                exchange(0, gm).wait_recv()
                out_ref[0, :, pl.ds(gm * GW, GW)] = (
                    out_ref[0, :, pl.ds(gm * GW, GW)] + recv_buf[gm]
                )

            return carry

        lax.fori_loop(0, NB, step, 0)

        exchange(0, NG - 1).wait_recv()
        out_ref[0, :, pl.ds((NG - 1) * GW, GW)] = (
            out_ref[0, :, pl.ds((NG - 1) * GW, GW)] + recv_buf[NG - 1]
        )
        for s in range(N_SEND_SLOTS):
            exchange(s, 0).wait_send()

    return pl.pallas_call(
        body,
        out_shape=jax.ShapeDtypeStruct((1, S, N), jnp.bfloat16),
        in_specs=[
            pl.BlockSpec(memory_space=pltpu.MemorySpace.VMEM),
            pl.BlockSpec(memory_space=pl.ANY),
        ],
        out_specs=pl.BlockSpec(memory_space=pltpu.MemorySpace.VMEM),
        scratch_shapes=[
            pltpu.VMEM((2, K, BN), jnp.float32),
            pltpu.VMEM((N_SEND_SLOTS, S, GW), jnp.bfloat16),
            pltpu.VMEM((NG, S, GW), jnp.bfloat16),
            pltpu.SemaphoreType.DMA((2,)),
            pltpu.SemaphoreType.DMA((N_SEND_SLOTS,)),
            pltpu.SemaphoreType.DMA((NG,)),
        ],
        compiler_params=pltpu.CompilerParams(
            collective_id=0,
            vmem_limit_bytes=64 * 1024 * 1024,
        ),
    )(A, Wo)


def kernel(O, Wo):
    B, S2, H, D = O.shape
    S = S2 // 2
    A = O.reshape(S2, H * D).astype(jnp.bfloat16)
    return _fused(A, Wo, S)


# device time: 254124 ns/iter; 1.4394x vs baseline; 1.4394x over previous
import jax
import jax.numpy as jnp
from jax import lax
from jax.experimental import pallas as pl
from jax.experimental.pallas import tpu as pltpu

BN = 256
G = 4
N_SEND_SLOTS = 2


def _fused(A, Wo, S):
    S2, K = A.shape
    N = Wo.shape[1]
    NB = N // BN
    NG = NB // G
    GW = G * BN

    def body(a_ref, w_hbm, out_ref, w_buf, send_buf, recv_buf,
             w_sems, send_sems, recv_sems):
        my_x = lax.axis_index("x")
        my_y = lax.axis_index("y")
        nbr = (my_x, 1 - my_y)
        keep_off = my_y * S
        send_off = (1 - my_y) * S

        def w_copy(j, slot):
            return pltpu.make_async_copy(
                w_hbm.at[:, pl.ds(j * BN, BN)], w_buf.at[slot], w_sems.at[slot]
            )

        def exchange(ssl, g):
            return pltpu.make_async_remote_copy(
                src_ref=send_buf.at[ssl],
                dst_ref=recv_buf.at[g],
                send_sem=send_sems.at[ssl],
                recv_sem=recv_sems.at[g],
                device_id=nbr,
                device_id_type=pl.DeviceIdType.MESH,
            )

        w_copy(0, 0).start()

        barrier = pltpu.get_barrier_semaphore()
        pl.semaphore_signal(
            barrier, inc=1, device_id=nbr, device_id_type=pl.DeviceIdType.MESH
        )
        pl.semaphore_wait(barrier, 1)

        def step(j, carry):
            wslot = lax.rem(j, 2)
            u = lax.rem(j, G)
            ssl = lax.rem(lax.div(j, G), N_SEND_SLOTS)

            @pl.when(j + 1 < NB)
            def _():
                w_copy(j + 1, lax.rem(j + 1, 2)).start()

            w_copy(j, wslot).wait()
            w = w_buf[wslot].astype(jnp.bfloat16)

            p_s = jax.lax.dot(
                a_ref[pl.ds(send_off, S), :], w,
                preferred_element_type=jnp.float32,
            )
            send_buf[ssl, :, pl.ds(u * BN, BN)] = p_s.astype(jnp.bfloat16)

            p_k = jax.lax.dot(
                a_ref[pl.ds(keep_off, S), :], w,
                preferred_element_type=jnp.float32,
            )
            out_ref[0, :, pl.ds(j * BN, BN)] = p_k.astype(jnp.bfloat16)
            return carry

        for g in range(NG):
            ssl = g % N_SEND_SLOTS
            if g >= N_SEND_SLOTS:
                exchange(ssl, 0).wait_send()
            lax.fori_loop(g * G, (g + 1) * G, step, 0)
            exchange(ssl, g).start()
            if g >= 1:
                gm = g - 1
                exchange(0, gm).wait_recv()
                out_ref[0, :, pl.ds(gm * GW, GW)] = (
                    out_ref[0, :, pl.ds(gm * GW, GW)] + recv_buf[gm]
                )

        exchange(0, NG - 1).wait_recv()
        out_ref[0, :, pl.ds((NG - 1) * GW, GW)] = (
            out_ref[0, :, pl.ds((NG - 1) * GW, GW)] + recv_buf[NG - 1]
        )
        for s in range(N_SEND_SLOTS):
            exchange(s, 0).wait_send()

    return pl.pallas_call(
        body,
        out_shape=jax.ShapeDtypeStruct((1, S, N), jnp.bfloat16),
        in_specs=[
            pl.BlockSpec(memory_space=pltpu.MemorySpace.VMEM),
            pl.BlockSpec(memory_space=pl.ANY),
        ],
        out_specs=pl.BlockSpec(memory_space=pltpu.MemorySpace.VMEM),
        scratch_shapes=[
            pltpu.VMEM((2, K, BN), jnp.float32),
            pltpu.VMEM((N_SEND_SLOTS, S, GW), jnp.bfloat16),
            pltpu.VMEM((NG, S, GW), jnp.bfloat16),
            pltpu.SemaphoreType.DMA((2,)),
            pltpu.SemaphoreType.DMA((N_SEND_SLOTS,)),
            pltpu.SemaphoreType.DMA((NG,)),
        ],
        compiler_params=pltpu.CompilerParams(
            collective_id=0,
            vmem_limit_bytes=64 * 1024 * 1024,
        ),
    )(A, Wo)


def kernel(O, Wo):
    B, S2, H, D = O.shape
    S = S2 // 2
    A = O.reshape(S2, H * D).astype(jnp.bfloat16)
    return _fused(A, Wo, S)
